# baseline (speedup 1.0000x reference)
import functools

import jax
import jax.numpy as jnp
from jax import lax
from jax.experimental import pallas as pl
from jax.experimental.pallas import tpu as pltpu
from jax.experimental.pallas import tpu_sc as plsc

NC = 2
NS = 16
NW = NC * NS
CH = 2048

IOTA = None


def _fused_kernel(n, m, mp):
    mesh = plsc.VectorSubcoreMesh(core_axis_name="c", subcore_axis_name="s")
    msl = mp // NS
    mh = mp // NW

    @functools.partial(
        pl.kernel,
        out_type=jax.ShapeDtypeStruct((mp,), jnp.float32),
        mesh=mesh,
        scratch_types=[
            [pltpu.VMEM((CH,), jnp.int32)] * 4,
            [pltpu.VMEM((CH,), jnp.int32)] * 4,
            [pltpu.VMEM((CH,), jnp.float32)] * 4,
            pltpu.VMEM((128,), jnp.float32),
            pltpu.VMEM(((NW + 1) * 16,), jnp.int32),
            pltpu.VMEM((msl,), jnp.float32),
            pltpu.VMEM_SHARED((mp,), jnp.float32),
            [pltpu.SemaphoreType.DMA] * 4,
            [pltpu.SemaphoreType.DMA] * 4,
        ],
        compiler_params=pltpu.CompilerParams(needs_layout_passes=False),
    )
    def fused(seg1, z1, tbl, er, bnd16, outp, segv, zv, ev, tblv, bndv, mbuf,
              acc, isem, ssem):
        cid = lax.axis_index("c")
        sid = lax.axis_index("s")
        w = cid * NS + sid

        pltpu.sync_copy(er.at[pl.ds(sid * msl, msl)], mbuf)
        pltpu.sync_copy(mbuf, acc.at[pl.ds(sid * msl, msl)])
        pltpu.sync_copy(tbl, tblv)
        pltpu.sync_copy(bnd16, bndv)
        plsc.subcore_barrier()

        b0 = jnp.max(bndv[pl.ds(w * 16, 16)])
        b1 = jnp.max(bndv[pl.ds(w * 16 + 16, 16)])
        sb = jnp.bitwise_and(b0, -8)
        se = jnp.bitwise_and(b1 + 7, -8)
        smax = jnp.maximum(0, se - CH)
        nch = (se - sb + CH - 1) // CH
        iota16 = lax.iota(jnp.int32, 16)
        pad16 = m + (w * 16 + iota16) % (mp - m)

        def chunk_start(ci):
            return pl.multiple_of(jnp.minimum(sb + ci * CH, smax), 8)

        def issue_inputs(ci, b):
            s = chunk_start(ci)
            pltpu.async_copy(seg1.at[pl.ds(s, CH)], segv[b], isem[b])
            pltpu.async_copy(z1.at[pl.ds(s, CH)], zv[b], isem[b])

        def wait_inputs(b):
            pltpu.make_async_copy(seg1.at[pl.ds(0, CH)], segv[b],
                                  isem[b]).wait()
            pltpu.make_async_copy(z1.at[pl.ds(0, CH)], zv[b],
                                  isem[b]).wait()

        def compute_and_scatter(ci, b):
            s = chunk_start(ci)
            lo = jnp.maximum(b0, sb + ci * CH)
            for v in range(CH // 16):
                o = v * 16
                idx = (s + o) + iota16
                sa = segv[b][pl.ds(o, 16)]
                za = zv[b][pl.ds(o, 16)]
                ev[b][pl.ds(o, 16)] = plsc.load_gather(tblv, [za])
                valid = jnp.logical_and(idx >= lo, idx < b1)
                segv[b][pl.ds(o, 16)] = jnp.where(valid, sa, pad16)
            pltpu.async_copy(ev[b], acc.at[segv[b]], ssem[b], add=True)

        def drain_scatter(b):
            pltpu.make_async_copy(ev[b], acc.at[segv[b]],
                                  ssem[b]).wait()

        @pl.when(nch > 0)
        def _():
            issue_inputs(0, 0)

        @pl.when(nch > 1)
        def _():
            issue_inputs(1, 1)

        nsteps = -(-(nch + 2) // 4) * 4

        def quad_body(t, _):
            for b in range(4):
                ci = t * 4 + b
                b2 = (b + 2) % 4

                @pl.when(ci < nch)
                def _(b=b):
                    wait_inputs(b)

                @pl.when(jnp.logical_and(ci - 2 >= 0, ci - 2 < nch))
                def _(b2=b2):
                    drain_scatter(b2)

                @pl.when(ci + 2 < nch)
                def _(ci=ci, b2=b2):
                    issue_inputs(ci + 2, b2)

                @pl.when(ci < nch)
                def _(ci=ci, b=b):
                    compute_and_scatter(ci, b)
            return 0

        lax.fori_loop(0, nsteps // 4, quad_body, 0)

        plsc.subcore_barrier()
        pltpu.sync_copy(acc.at[pl.ds(w * mh, mh)], mbuf.at[pl.ds(0, mh)])
        pltpu.sync_copy(mbuf.at[pl.ds(0, mh)], outp.at[pl.ds(w * mh, mh)])

    return fused


def kernel(energy_readout, atomic_numbers, atomic_subsystem_indices,
           self_energies_tensor):
    m = energy_readout.shape[0]
    n = atomic_numbers.shape[0]
    mp = -(-m // 512) * 512
    if mp == m:
        mp += 512

    seg = atomic_subsystem_indices.astype(jnp.int32)
    z = atomic_numbers.astype(jnp.int32)
    if n % 8:
        npad = 8 - n % 8
        seg = jnp.concatenate([seg, jnp.full((npad,), m, jnp.int32)])
        z = jnp.concatenate([z, jnp.zeros((npad,), jnp.int32)])
        n += npad
    tbl16 = jnp.zeros((128,), jnp.float32).at[: self_energies_tensor.shape[0]].set(
        self_energies_tensor.astype(jnp.float32))
    er_p = jnp.zeros((mp,), jnp.float32).at[:m].set(
        energy_readout.astype(jnp.float32))
    mh = mp // NW
    bnd = jnp.searchsorted(
        seg, jnp.arange(NW + 1, dtype=jnp.int32) * mh, side="left"
    ).astype(jnp.int32)
    bnd16 = jnp.repeat(bnd, 16)

    outp = _fused_kernel(n, m, mp)(seg, z, tbl16, er_p, bnd16)
    return outp[:m]
